# baseline (speedup 1.0000x reference)
import jax
import jax.numpy as jnp
import numpy as np
from jax import lax
from jax.experimental import pallas as pl
from jax.experimental.pallas import tpu as pltpu
from jax.experimental.pallas import tpu_sc as plsc

N_VOCAB = 100000
EMBED_DIM = 768
BATCH = 4
SEQ_LEN = 8192

NUM_WORKERS = 32
POS_PER_WORKER = SEQ_LEN // NUM_WORKERS
CHUNK = 32
N_CHUNKS = POS_PER_WORKER // CHUNK
N_STEPS = N_CHUNKS * BATCH
LANES = 16
VECS_PER_ROW = EMBED_DIM // LANES


def _positional_encoding_np(max_len, d):
    pos = np.arange(max_len, dtype=np.float64)[:, None]
    i = np.arange(0, d, 2, dtype=np.float64)
    div = np.exp(-(np.log(10000.0) * i / d))
    ang = pos * div[None, :]
    pe = np.zeros((max_len, d), dtype=np.float64)
    pe[:, 0::2] = np.sin(ang)
    pe[:, 1::2] = np.cos(ang)
    return pe.astype(np.float32)


_PE = _positional_encoding_np(SEQ_LEN, EMBED_DIM)


def _sc_body(x_hbm, table_hbm, pe_hbm, out_hbm,
             idx_all, pe_v, rows_v, g0, g1, s0, s1, p0, p1):
    gsem = [g0, g1]
    ssem = [s0, s1]
    pesem = [p0, p1]
    wid = lax.axis_index("s") * 2 + lax.axis_index("c")
    pos0 = wid * POS_PER_WORKER

    for b in range(BATCH):
        pltpu.sync_copy(x_hbm.at[pl.ds(b * SEQ_LEN + pos0, POS_PER_WORKER)],
                        idx_all.at[b])

    g_obj = [None, None]
    s_obj = [None, None]
    pe_obj = [None, None]

    def start(t):
        j, b = t // BATCH, t % BATCH
        par = t % 2
        if b == 0:
            pe_obj[j % 2] = pltpu.async_copy(
                pe_hbm.at[pl.ds(pos0 + j * CHUNK, CHUNK)],
                pe_v.at[j % 2], pesem[j % 2])
        if s_obj[par] is not None:
            s_obj[par].wait()
            s_obj[par] = None
        g_obj[par] = pltpu.async_copy(
            table_hbm.at[idx_all.at[b, pl.ds(j * CHUNK, CHUNK)]],
            rows_v.at[par], gsem[par])

    def finish(t):
        j, b = t // BATCH, t % BATCH
        par = t % 2
        pj = j % 2
        if b == 0 and pe_obj[pj] is not None:
            pe_obj[pj].wait()
            pe_obj[pj] = None
        g_obj[par].wait()
        g_obj[par] = None

        def add_row(r, c):
            for k in range(VECS_PER_ROW):
                sl = pl.ds(k * LANES, LANES)
                rows_v[par, r, sl] = rows_v[par, r, sl] + pe_v[pj, r, sl]
            return c

        lax.fori_loop(0, CHUNK, add_row, 0)
        base = b * SEQ_LEN + pos0 + j * CHUNK
        s_obj[par] = pltpu.async_copy(rows_v.at[par],
                                      out_hbm.at[pl.ds(base, CHUNK)],
                                      ssem[par])

    start(0)
    for t in range(N_STEPS):
        if t + 1 < N_STEPS:
            start(t + 1)
        finish(t)
    s_obj[0].wait()
    s_obj[1].wait()


def kernel(x, token_table):
    x_flat = x.reshape(-1).astype(jnp.int32)
    pe = jnp.asarray(_PE)

    mesh = plsc.VectorSubcoreMesh(core_axis_name="c", subcore_axis_name="s")
    run = pl.kernel(
        _sc_body,
        out_type=jax.ShapeDtypeStruct((BATCH * SEQ_LEN, EMBED_DIM), jnp.float32),
        mesh=mesh,
        scratch_types=[
            pltpu.VMEM((BATCH, POS_PER_WORKER), jnp.int32),
            pltpu.VMEM((2, CHUNK, EMBED_DIM), jnp.float32),
            pltpu.VMEM((2, CHUNK, EMBED_DIM), jnp.float32),
            pltpu.SemaphoreType.DMA,
            pltpu.SemaphoreType.DMA,
            pltpu.SemaphoreType.DMA,
            pltpu.SemaphoreType.DMA,
            pltpu.SemaphoreType.DMA,
            pltpu.SemaphoreType.DMA,
        ],
    )
    out = run(x_flat, token_table, pe)
    return out.reshape(BATCH, SEQ_LEN, EMBED_DIM)
